# baseline (speedup 1.0000x reference)
import functools

import jax
import jax.numpy as jnp
from jax import lax
from jax.experimental import pallas as pl
from jax.experimental.pallas import tpu as pltpu
from jax.experimental.pallas import tpu_sc as plsc

HIDDEN = 64
LANES = 16
NCORES = 2
NSUBCORES = 16
NW = NCORES * NSUBCORES

SUB = 128
NSUB = 4
CHUNK = SUB * NSUB


_GATHER_DNUMS = lax.GatherDimensionNumbers(
    offset_dims=(), collapsed_slice_dims=(0,), start_index_map=(0,)
)


def _perm(v, idx16):
    return lax.gather(
        v,
        idx16[:, None],
        _GATHER_DNUMS,
        slice_sizes=(1,),
        mode=lax.GatherScatterMode.PROMISE_IN_BOUNDS,
    )


def _rsqrt(s):
    i = lax.bitcast_convert_type(s, jnp.int32)
    i = jnp.int32(0x5F3759DF) - lax.shift_right_logical(i, 1)
    y = lax.bitcast_convert_type(i, jnp.float32)
    for _ in range(3):
        y = y * (1.5 - 0.5 * s * y * y)
    return y


def _make_kernel(total_rows):
    per_w = total_rows // NW
    n_chunks = per_w // CHUNK
    mesh = plsc.VectorSubcoreMesh(core_axis_name="c", subcore_axis_name="s")

    @functools.partial(
        pl.kernel,
        mesh=mesh,
        out_type=jax.ShapeDtypeStruct((total_rows, HIDDEN), jnp.float32),
        scratch_types=[
            pltpu.VMEM((NSUB, SUB), jnp.int32),
            pltpu.VMEM((CHUNK, HIDDEN), jnp.float32),
            pltpu.SemaphoreType.DMA,
        ],
        compiler_params=pltpu.CompilerParams(use_tc_tiling_on_sc=False),
    )
    def gather_norm(idx_hbm, table_hbm, out_hbm, idx_v, rows_v, gsem):
        wid = lax.axis_index("s") * NCORES + lax.axis_index("c")
        base = wid * per_w
        irow0 = wid * (per_w // SUB)
        lane = lax.iota(jnp.int32, LANES)
        perms = [lane ^ (1 << k) for k in (3, 2, 1, 0)]

        def chunk_body(g, carry):
            pltpu.sync_copy(idx_hbm.at[pl.ds(irow0 + g * NSUB, NSUB)], idx_v)
            copies = [
                pltpu.async_copy(
                    table_hbm.at[idx_v.at[j]],
                    rows_v.at[pl.ds(j * SUB, SUB)],
                    gsem,
                )
                for j in range(NSUB)
            ]
            for c in copies:
                c.wait()

            def row_body(i, _):
                v0 = rows_v[i, pl.ds(0, LANES)]
                v1 = rows_v[i, pl.ds(LANES, LANES)]
                v2 = rows_v[i, pl.ds(2 * LANES, LANES)]
                v3 = rows_v[i, pl.ds(3 * LANES, LANES)]
                q = v0 * v0 + v1 * v1 + v2 * v2 + v3 * v3
                for p in perms:
                    q = q + _perm(q, p)
                r = _rsqrt(q)
                rows_v[i, pl.ds(0, LANES)] = v0 * r
                rows_v[i, pl.ds(LANES, LANES)] = v1 * r
                rows_v[i, pl.ds(2 * LANES, LANES)] = v2 * r
                rows_v[i, pl.ds(3 * LANES, LANES)] = v3 * r
                return 0

            lax.fori_loop(0, CHUNK, row_body, 0)
            pltpu.sync_copy(rows_v, out_hbm.at[pl.ds(base + g * CHUNK, CHUNK)])
            return carry

        lax.fori_loop(0, n_chunks, chunk_body, 0)

    return gather_norm


def kernel(inputs, table):
    total = inputs.size
    idx2 = inputs.reshape(total // SUB, SUB)
    out = _make_kernel(total)(idx2, table)
    return out.reshape(inputs.shape + (HIDDEN,))
